# baseline (speedup 1.0000x reference)
import functools

import jax
import jax.numpy as jnp
from jax import lax
from jax.experimental import pallas as pl
from jax.experimental.pallas import tpu as pltpu
from jax.experimental.pallas import tpu_sc as plsc

_NS = 20
_NBLK = 128
_D = 32
_V = 1000000
_FULL_BLOCKS = _V // 128
_TAIL_ROWS = _V - _FULL_BLOCKS * 128


@functools.cache
def _make_transpose():
    info = plsc.get_sparse_core_info()
    nc = info.num_cores
    nw = nc * info.num_subcores
    base_blocks = _FULL_BLOCKS // nw
    extra = _FULL_BLOCKS - base_blocks * nw
    mesh = plsc.VectorSubcoreMesh(core_axis_name="c", subcore_axis_name="s")

    @functools.partial(
        pl.kernel,
        mesh=mesh,
        compiler_params=pltpu.CompilerParams(needs_layout_passes=False),
        out_type=jax.ShapeDtypeStruct((_V * _D,), jnp.float32),
        scratch_types=[
            pltpu.VMEM((2, _D, 128), jnp.float32),
            pltpu.VMEM((_D * 128,), jnp.float32),
            pltpu.VMEM((_D * 128,), jnp.float32),
            pltpu.SemaphoreType.DMA,
            pltpu.SemaphoreType.DMA,
            pltpu.SemaphoreType.DMA,
            pltpu.SemaphoreType.DMA,
        ],
    )
    def tr_kernel(wt_hbm, tail_hbm, out_hbm, src_v, dst0, dst1, gA, gB, sA, sB):
        dsts = (dst0, dst1)
        wid = lax.axis_index("s") * nc + lax.axis_index("c")
        gsems = (gA, gB)
        ssems = (sA, sB)
        lane = lax.iota(jnp.int32, 16)

        def blk_of(i):
            return wid + i * nw

        def issue(i, p):
            pltpu.async_copy(
                wt_hbm.at[:, pl.ds(blk_of(i) * 128, 128)], src_v.at[p], gsems[p]
            )

        def repack(p, ncols):
            for tb in range(ncols // 16):
                tsel = lane + tb * 16

                def dbody(k, carry, _p=p, _tsel=tsel):
                    for u in range(8):
                        d0 = k * 8 + u
                        dcol = (d0 + lane) & 31
                        vals = plsc.load_gather(src_v.at[_p], [dcol, _tsel])
                        plsc.store_scatter(dsts[_p], [_tsel * 32 + dcol], vals)
                    return carry

                lax.fori_loop(0, _D // 8, dbody, 0)

        def store(i, p):
            pltpu.async_copy(
                dsts[p], out_hbm.at[pl.ds(blk_of(i) * 4096, _D * 128)], ssems[p]
            )

        def drain_store(p):
            pltpu.make_async_copy(
                out_hbm.at[pl.ds(0, _D * 128)], dsts[p], ssems[p]
            ).wait()

        def drain_gather(p):
            pltpu.make_async_copy(
                wt_hbm.at[:, pl.ds(0, 128)], src_v.at[p], gsems[p]
            ).wait()

        nblk = base_blocks + jnp.where(wid < extra, 1, 0)

        issue(0, 0)

        def body(j, carry):
            for p in range(2):
                i = j * 2 + p

                @pl.when(i < nblk)
                def _():
                    @pl.when(i + 1 < nblk)
                    def _():
                        issue(i + 1, 1 - p)

                    drain_gather(p)

                    @pl.when(i >= 2)
                    def _():
                        drain_store(p)

                    repack(p, 128)
                    store(i, p)

            return carry

        lax.fori_loop(0, (base_blocks + 2) // 2, body, 0)
        for p in range(2):
            @pl.when(nblk >= p + 1)
            def _():
                drain_store(p)

        @pl.when(wid == nw - 1)
        def _():
            pltpu.sync_copy(tail_hbm, dst0.at[pl.ds(0, 2048)])
            pltpu.sync_copy(
                dst0.at[pl.ds(0, 2048)],
                out_hbm.at[pl.ds(_FULL_BLOCKS * 4096, 2048)],
            )

    return tr_kernel


@functools.cache
def _make_gather():
    info = plsc.get_sparse_core_info()
    nc = info.num_cores
    nw = nc * info.num_subcores
    n_groups = (_NBLK // nw) * _NS
    mesh = plsc.VectorSubcoreMesh(core_axis_name="c", subcore_axis_name="s")

    @functools.partial(
        pl.kernel,
        mesh=mesh,
        compiler_params=pltpu.CompilerParams(
            use_tc_tiling_on_sc=False, needs_layout_passes=False
        ),
        out_type=jax.ShapeDtypeStruct((_NS, 4, _NBLK, 8, 128), jnp.float32),
        scratch_types=[
            pltpu.VMEM((4, _NS, 128), jnp.int32),
            pltpu.VMEM((128, _D), jnp.float32),
            pltpu.VMEM((128, _D), jnp.float32),
            pltpu.VMEM((_D, 128), jnp.float32),
            pltpu.VMEM((_D, 128), jnp.float32),
            pltpu.SemaphoreType.DMA,
            pltpu.SemaphoreType.DMA,
            pltpu.SemaphoreType.DMA,
            pltpu.SemaphoreType.DMA,
        ],
    )
    def gather_kernel(ids3_hbm, w_hbm, out_hbm, idx_v, rA, rB, cA, cB, gA, gB, sA, sB):
        wid = lax.axis_index("s") * nc + lax.axis_index("c")
        bbase = wid * 4
        pltpu.sync_copy(ids3_hbm.at[pl.ds(bbase, 4)], idx_v)

        rows = (rA, rB)
        chunks = (cA, cB)
        gsems = (gA, gB)
        ssems = (sA, sB)

        def issue(g, rbuf, sem):
            c = g // _NS
            s = g % _NS
            pltpu.async_copy(w_hbm.at[idx_v.at[c, s]], rbuf, sem)

        issue(0, rA, gA)

        def assemble(rbuf, cbuf):
            lane = lax.iota(jnp.int32, 16)
            for t0 in range(8):
                rowsel = lane + t0 * 16

                def dbody(k, carry, _rbuf=rbuf, _cbuf=cbuf, _rowsel=rowsel):
                    for u in range(4):
                        dcol = (k * 4 + u + lane) & 31
                        vals = plsc.load_gather(_rbuf, [_rowsel, dcol])
                        plsc.store_scatter(_cbuf, [dcol, _rowsel], vals)
                    return carry

                lax.fori_loop(0, _D // 4, dbody, 0)

        def half(j, p, goff):
            g = j * 2 + goff
            c = g // _NS
            s = g % _NS

            @pl.when(g + 1 < n_groups)
            def _():
                issue(g + 1, rows[1 - p], gsems[1 - p])

            pltpu.make_async_copy(
                w_hbm.at[idx_v.at[0, 0]], rows[p], gsems[p]
            ).wait()

            @pl.when(j > 0)
            def _():
                for dblk in range(4):
                    pltpu.make_async_copy(
                        out_hbm.at[0, 0, 0],
                        chunks[p].at[pl.ds(dblk * 8, 8)],
                        ssems[p],
                    ).wait()

            assemble(rows[p], chunks[p])
            for dblk in range(4):
                pltpu.async_copy(
                    chunks[p].at[pl.ds(dblk * 8, 8)],
                    out_hbm.at[s, dblk, bbase + c],
                    ssems[p],
                )

        def body(j, carry):
            half(j, 0, 0)
            half(j, 1, 1)
            return carry

        lax.fori_loop(0, n_groups // 2, body, 0)
        for p in range(2):
            for dblk in range(4):
                pltpu.make_async_copy(
                    out_hbm.at[0, 0, 0],
                    chunks[p].at[pl.ds(dblk * 8, 8)],
                    ssems[p],
                ).wait()

    return gather_kernel


@jax.jit
def kernel(token_ids, weights):
    tailflat = weights[_FULL_BLOCKS * 128 :].reshape(2048)
    wflat = _make_transpose()(weights.T, tailflat)
    wlin = wflat.reshape(_V, _D)
    ids3 = token_ids.T.reshape(_NS, _NBLK, 128).transpose(1, 0, 2).astype(jnp.int32)
    n = _make_gather()(ids3, wlin)
    return n.transpose(2, 4, 0, 1, 3).reshape(16384, _NS, _D)


# speedup vs baseline: 1.2495x; 1.2495x over previous
import functools

import jax
import jax.numpy as jnp
from jax import lax
from jax.experimental import pallas as pl
from jax.experimental.pallas import tpu as pltpu
from jax.experimental.pallas import tpu_sc as plsc

_NS = 20
_NBLK = 128
_D = 32
_V = 1000000
_FULL_BLOCKS = _V // 128
_TAIL_ROWS = _V - _FULL_BLOCKS * 128


@functools.cache
def _make_transpose():
    info = plsc.get_sparse_core_info()
    nc = info.num_cores
    nw = nc * info.num_subcores
    base_blocks = _FULL_BLOCKS // nw
    extra = _FULL_BLOCKS - base_blocks * nw
    mesh = plsc.VectorSubcoreMesh(core_axis_name="c", subcore_axis_name="s")

    @functools.partial(
        pl.kernel,
        mesh=mesh,
        compiler_params=pltpu.CompilerParams(needs_layout_passes=False),
        out_type=jax.ShapeDtypeStruct((_V // 4, 128), jnp.float32),
        scratch_types=[
            pltpu.VMEM((2, _D, 128), jnp.float32),
            pltpu.VMEM((2, _D, 128), jnp.float32),
            pltpu.SemaphoreType.DMA,
            pltpu.SemaphoreType.DMA,
            pltpu.SemaphoreType.DMA,
            pltpu.SemaphoreType.DMA,
        ],
    )
    def tr_kernel(wt_hbm, tail_hbm, out_hbm, src_v, dst_v, gA, gB, sA, sB):
        wid = lax.axis_index("s") * nc + lax.axis_index("c")
        gsems = (gA, gB)
        ssems = (sA, sB)
        lane = lax.iota(jnp.int32, 16)

        def blk_of(i):
            return wid + i * nw

        def issue(i, p):
            pltpu.async_copy(
                wt_hbm.at[:, pl.ds(blk_of(i) * 128, 128)], src_v.at[p], gsems[p]
            )

        def repack(p, ncols):
            for tb in range(ncols // 16):
                tsel = lane + tb * 16

                def dbody(k, carry, _p=p, _tsel=tsel):
                    for u in range(4):
                        d0 = k * 4 + u
                        dcol = (d0 + lane) & 31
                        vals = plsc.load_gather(src_v.at[_p], [dcol, _tsel])
                        q = _tsel * 32 + dcol
                        plsc.store_scatter(
                            dst_v.at[_p],
                            [lax.shift_right_logical(q, 7), q & 127],
                            vals,
                        )
                    return carry

                lax.fori_loop(0, _D // 4, dbody, 0)

        def store(i, p):
            pltpu.async_copy(
                dst_v.at[p], out_hbm.at[pl.ds(blk_of(i) * 32, _D)], ssems[p]
            )

        def drain_store(p):
            pltpu.make_async_copy(
                out_hbm.at[pl.ds(0, _D)], dst_v.at[p], ssems[p]
            ).wait()

        def drain_gather(p):
            pltpu.make_async_copy(
                out_hbm.at[pl.ds(0, _D)], src_v.at[p], gsems[p]
            ).wait()

        nblk = base_blocks + jnp.where(wid < extra, 1, 0)

        issue(0, 0)

        def body(j, carry):
            for p in range(2):
                i = j * 2 + p

                @pl.when(i < nblk)
                def _():
                    @pl.when(i + 1 < nblk)
                    def _():
                        issue(i + 1, 1 - p)

                    drain_gather(p)

                    @pl.when(i >= 2)
                    def _():
                        drain_store(p)

                    repack(p, 128)
                    store(i, p)

            return carry

        lax.fori_loop(0, (base_blocks + 2) // 2, body, 0)
        for p in range(2):
            @pl.when(nblk >= p + 1)
            def _():
                drain_store(p)

        @pl.when(wid == nw - 1)
        def _():
            pltpu.sync_copy(tail_hbm, dst_v.at[0, pl.ds(0, 16)])
            pltpu.sync_copy(
                dst_v.at[0, pl.ds(0, 16)],
                out_hbm.at[pl.ds(_FULL_BLOCKS * 32, 16)],
            )

    return tr_kernel


@functools.cache
def _make_gather():
    info = plsc.get_sparse_core_info()
    nc = info.num_cores
    nw = nc * info.num_subcores
    n_groups = (_NBLK // nw) * _NS
    mesh = plsc.VectorSubcoreMesh(core_axis_name="c", subcore_axis_name="s")

    @functools.partial(
        pl.kernel,
        mesh=mesh,
        compiler_params=pltpu.CompilerParams(
            use_tc_tiling_on_sc=False, needs_layout_passes=False
        ),
        out_type=jax.ShapeDtypeStruct((_NS, 4, _NBLK, 8, 128), jnp.float32),
        scratch_types=[
            pltpu.VMEM((4, _NS, 128), jnp.int32),
            pltpu.VMEM((128, _D), jnp.float32),
            pltpu.VMEM((128, _D), jnp.float32),
            pltpu.VMEM((_D, 128), jnp.float32),
            pltpu.VMEM((_D, 128), jnp.float32),
            pltpu.SemaphoreType.DMA,
            pltpu.SemaphoreType.DMA,
            pltpu.SemaphoreType.DMA,
            pltpu.SemaphoreType.DMA,
        ],
    )
    def gather_kernel(ids3_hbm, w_hbm, out_hbm, idx_v, rA, rB, cA, cB, gA, gB, sA, sB):
        wid = lax.axis_index("s") * nc + lax.axis_index("c")
        bbase = wid * 4
        pltpu.sync_copy(ids3_hbm.at[pl.ds(bbase, 4)], idx_v)

        rows = (rA, rB)
        chunks = (cA, cB)
        gsems = (gA, gB)
        ssems = (sA, sB)

        def issue(g, rbuf, sem):
            c = g // _NS
            s = g % _NS
            pltpu.async_copy(w_hbm.at[idx_v.at[c, s]], rbuf, sem)

        issue(0, rA, gA)

        def assemble(rbuf, cbuf):
            lane = lax.iota(jnp.int32, 16)
            for t0 in range(8):
                rowsel = lane + t0 * 16

                def dbody(k, carry, _rbuf=rbuf, _cbuf=cbuf, _rowsel=rowsel):
                    for u in range(4):
                        dcol = (k * 4 + u + lane) & 31
                        vals = plsc.load_gather(_rbuf, [_rowsel, dcol])
                        plsc.store_scatter(_cbuf, [dcol, _rowsel], vals)
                    return carry

                lax.fori_loop(0, _D // 4, dbody, 0)

        def half(j, p, goff):
            g = j * 2 + goff
            c = g // _NS
            s = g % _NS

            @pl.when(g + 1 < n_groups)
            def _():
                issue(g + 1, rows[1 - p], gsems[1 - p])

            pltpu.make_async_copy(
                w_hbm.at[idx_v.at[0, 0]], rows[p], gsems[p]
            ).wait()

            @pl.when(j > 0)
            def _():
                for dblk in range(4):
                    pltpu.make_async_copy(
                        out_hbm.at[0, 0, 0],
                        chunks[p].at[pl.ds(dblk * 8, 8)],
                        ssems[p],
                    ).wait()

            assemble(rows[p], chunks[p])
            for dblk in range(4):
                pltpu.async_copy(
                    chunks[p].at[pl.ds(dblk * 8, 8)],
                    out_hbm.at[s, dblk, bbase + c],
                    ssems[p],
                )

        def body(j, carry):
            half(j, 0, 0)
            half(j, 1, 1)
            return carry

        lax.fori_loop(0, n_groups // 2, body, 0)
        for p in range(2):
            for dblk in range(4):
                pltpu.make_async_copy(
                    out_hbm.at[0, 0, 0],
                    chunks[p].at[pl.ds(dblk * 8, 8)],
                    ssems[p],
                ).wait()

    return gather_kernel


@jax.jit
def kernel(token_ids, weights):
    tailflat = weights[_FULL_BLOCKS * 128 :].reshape(16, 128)
    w128 = _make_transpose()(weights.T, tailflat)
    wlin = w128.reshape(_V, _D)
    ids3 = token_ids.T.reshape(_NS, _NBLK, 128).transpose(1, 0, 2).astype(jnp.int32)
    n = _make_gather()(ids3, wlin)
    return n.transpose(2, 4, 0, 1, 3).reshape(16384, _NS, _D)
